# baseline (speedup 1.0000x reference)
import functools

import jax
import jax.numpy as jnp
from jax import lax
from jax.experimental import pallas as pl
from jax.experimental.pallas import tpu as pltpu
from jax.experimental.pallas import tpu_sc as plsc


def _gather_kernel(B0, B1, D, R):
    info = plsc.get_sparse_core_info()
    NC, NS = info.num_cores, info.num_subcores
    NW = NC * NS
    rows_per_w = B0 // NW
    n_chunks = rows_per_w // R
    assert n_chunks % 2 == 0
    mesh = plsc.VectorSubcoreMesh(core_axis_name="c", subcore_axis_name="s")

    @functools.partial(
        pl.kernel,
        mesh=mesh,
        out_type=jax.ShapeDtypeStruct((B0, B1, D), jnp.float32),
        scratch_types=[
            pltpu.VMEM((R, B1), jnp.int32),
            pltpu.VMEM((R, B1), jnp.int32),
            pltpu.VMEM((R, B1, D), jnp.float32),
            pltpu.VMEM((R, B1, D), jnp.float32),
            pltpu.SemaphoreType.DMA,
            pltpu.SemaphoreType.DMA,
        ],
        compiler_params=pltpu.CompilerParams(use_tc_tiling_on_sc=False),
    )
    def k(table_hbm, data_hbm, out_hbm, raw0, raw1, rows0, rows1, sem0, sem1):
        wid = lax.axis_index("s") * NC + lax.axis_index("c")
        base = wid * rows_per_w
        bufs = ((raw0, rows0, sem0), (raw1, rows1, sem1))

        def start(g, b):
            raw, rows, sem = bufs[b]
            r0 = base + g * R
            pltpu.sync_copy(data_hbm.at[pl.ds(r0, R), :], raw)
            for j in range(R):
                pltpu.async_copy(table_hbm.at[raw.at[j]], rows.at[j], sem)

        def finish(g, b):
            raw, rows, sem = bufs[b]
            r0 = base + g * R
            for j in range(R):
                pltpu.make_async_copy(table_hbm.at[raw.at[j]], rows.at[j], sem).wait()
            pltpu.sync_copy(rows, out_hbm.at[pl.ds(r0, R), :, :])

        start(0, 0)

        def pair(j, carry):
            g = 2 * j
            start(g + 1, 1)
            finish(g, 0)

            @pl.when(g + 2 < n_chunks)
            def _():
                start(g + 2, 0)

            finish(g + 1, 1)
            return carry

        lax.fori_loop(0, n_chunks // 2, pair, 0)

    return k


def kernel(data, table):
    B0, B1 = data.shape
    V, D = table.shape
    idx = data.astype(jnp.int32)
    return _gather_kernel(B0, B1, D, 16)(table, idx)


# speedup vs baseline: 1.3459x; 1.3459x over previous
import functools

import jax
import jax.numpy as jnp
from jax import lax
from jax.experimental import pallas as pl
from jax.experimental.pallas import tpu as pltpu
from jax.experimental.pallas import tpu_sc as plsc


def _gather_kernel(B0, B1, D, R, B1P, DP):
    info = plsc.get_sparse_core_info()
    NC, NS = info.num_cores, info.num_subcores
    NW = NC * NS
    rows_per_w = B0 // NW
    n_chunks = rows_per_w // R
    assert n_chunks % 2 == 0
    mesh = plsc.VectorSubcoreMesh(core_axis_name="c", subcore_axis_name="s")

    @functools.partial(
        pl.kernel,
        mesh=mesh,
        out_type=jax.ShapeDtypeStruct((B0, B1P, DP), jnp.float32),
        scratch_types=[
            pltpu.VMEM((R, B1), jnp.int32),
            pltpu.VMEM((R, B1), jnp.int32),
            pltpu.VMEM((R, B1, D), jnp.float32),
            pltpu.VMEM((R, B1, D), jnp.float32),
            pltpu.SemaphoreType.DMA,
            pltpu.SemaphoreType.DMA,
        ],
        compiler_params=pltpu.CompilerParams(use_tc_tiling_on_sc=False),
    )
    def k(table_hbm, data_hbm, out_hbm, raw0, raw1, rows0, rows1, sem0, sem1):
        wid = lax.axis_index("s") * NC + lax.axis_index("c")
        base = wid * rows_per_w
        bufs = ((raw0, rows0, sem0), (raw1, rows1, sem1))

        def start(g, b):
            raw, rows, sem = bufs[b]
            r0 = base + g * R
            pltpu.sync_copy(data_hbm.at[pl.ds(r0, R), :], raw)
            for j in range(R):
                pltpu.async_copy(table_hbm.at[raw.at[j]], rows.at[j], sem)

        def finish(g, b):
            raw, rows, sem = bufs[b]
            r0 = base + g * R
            for j in range(R):
                pltpu.make_async_copy(table_hbm.at[raw.at[j]], rows.at[j], sem).wait()
            pltpu.sync_copy(
                rows, out_hbm.at[pl.ds(r0, R), pl.ds(0, B1), pl.ds(0, D)]
            )

        start(0, 0)

        def pair(j, carry):
            g = 2 * j
            start(g + 1, 1)
            finish(g, 0)

            @pl.when(g + 2 < n_chunks)
            def _():
                start(g + 2, 0)

            finish(g + 1, 1)
            return carry

        lax.fori_loop(0, n_chunks // 2, pair, 0)

    return k


def kernel(data, table):
    B0, B1 = data.shape
    V, D = table.shape
    idx = data.astype(jnp.int32)
    B1P = (B1 + 7) // 8 * 8
    DP = 128
    out = _gather_kernel(B0, B1, D, 16, B1P, DP)(table, idx)
    return out[:, :B1, :D]


# speedup vs baseline: 1.3530x; 1.0052x over previous
import functools

import jax
import jax.numpy as jnp
from jax import lax
from jax.experimental import pallas as pl
from jax.experimental.pallas import tpu as pltpu
from jax.experimental.pallas import tpu_sc as plsc


def _gather_kernel(B0, B1, D, R, B1P, DP):
    info = plsc.get_sparse_core_info()
    NC, NS = info.num_cores, info.num_subcores
    NW = NC * NS
    rows_per_w = B0 // NW
    n_chunks = rows_per_w // R
    assert n_chunks % 2 == 0
    mesh = plsc.VectorSubcoreMesh(core_axis_name="c", subcore_axis_name="s")

    @functools.partial(
        pl.kernel,
        mesh=mesh,
        out_type=jax.ShapeDtypeStruct((B0, B1P, DP), jnp.float32),
        scratch_types=[
            pltpu.VMEM((R, B1), jnp.int32),
            pltpu.VMEM((R, B1), jnp.int32),
            pltpu.VMEM((R, B1, D), jnp.float32),
            pltpu.VMEM((R, B1, D), jnp.float32),
            pltpu.SemaphoreType.DMA,
            pltpu.SemaphoreType.DMA,
            pltpu.SemaphoreType.DMA,
            pltpu.SemaphoreType.DMA,
        ],
        compiler_params=pltpu.CompilerParams(use_tc_tiling_on_sc=False),
    )
    def k(table_hbm, data_hbm, out_hbm,
          raw0, raw1, rows0, rows1, gsem0, gsem1, isem0, isem1):
        wid = lax.axis_index("s") * NC + lax.axis_index("c")
        base = wid * rows_per_w
        bufs = ((raw0, rows0, gsem0, isem0), (raw1, rows1, gsem1, isem1))

        def idx_start(g, b):
            raw, _, _, isem = bufs[b]
            pltpu.async_copy(data_hbm.at[pl.ds(base + g * R, R), :], raw, isem)

        def gathers(g, b):
            raw, rows, gsem, isem = bufs[b]
            pltpu.make_async_copy(
                data_hbm.at[pl.ds(base + g * R, R), :], raw, isem
            ).wait()
            for j in range(R):
                pltpu.async_copy(table_hbm.at[raw.at[j]], rows.at[j], gsem)

        def finish(g, b):
            raw, rows, gsem, isem = bufs[b]
            r0 = base + g * R
            for j in range(R):
                pltpu.make_async_copy(table_hbm.at[raw.at[j]], rows.at[j], gsem).wait()

            @pl.when(g + 2 < n_chunks)
            def _():
                idx_start(g + 2, b)

            pltpu.sync_copy(
                rows, out_hbm.at[pl.ds(r0, R), pl.ds(0, B1), pl.ds(0, D)]
            )

        idx_start(0, 0)
        idx_start(1, 1)
        gathers(0, 0)

        def pair(j, carry):
            g = 2 * j
            gathers(g + 1, 1)
            finish(g, 0)

            @pl.when(g + 2 < n_chunks)
            def _():
                gathers(g + 2, 0)

            finish(g + 1, 1)
            return carry

        lax.fori_loop(0, n_chunks // 2, pair, 0)

    return k


def kernel(data, table):
    B0, B1 = data.shape
    V, D = table.shape
    idx = data.astype(jnp.int32)
    B1P = (B1 + 7) // 8 * 8
    DP = 128
    out = _gather_kernel(B0, B1, D, 16, B1P, DP)(table, idx)
    return out[:, :B1, :D]
